# baseline (speedup 1.0000x reference)
import jax
import jax.numpy as jnp
from jax.experimental import pallas as pl
import jax.experimental.pallas.tpu as pltpu
import jax.experimental.pallas.tpu_sc as plsc

N_K = 1024
D = 256
B = 16384
BT = 1024
NCH = 4
CH = B // NCH
GW = 128


def _argmin_body(z_ref, emb_ref, idx_ref):
    z = z_ref[...]
    emb = emb_ref[...]
    zsq = jnp.sum(z * z, axis=-1, keepdims=True)
    esq = jnp.sum(emb * emb, axis=-1)
    zb2 = z.astype(jnp.bfloat16)
    zb2 = zb2 + zb2
    s2 = jax.lax.dot_general(
        zb2, emb.astype(jnp.bfloat16),
        (((1,), (1,)), ((), ())),
        preferred_element_type=jnp.float32)
    d = (zsq + esq[None, :]) - s2
    m = jnp.min(d, axis=-1, keepdims=True)
    iota = jax.lax.broadcasted_iota(jnp.int32, d.shape, 1)
    idx_ref[...] = jnp.min(jnp.where(d == m, iota, N_K), axis=-1)


def _tc_argmin(z, emb):
    n = z.shape[0]
    return pl.pallas_call(
        _argmin_body,
        grid=(n // BT,),
        in_specs=[
            pl.BlockSpec((BT, D), lambda i: (i, 0)),
            pl.BlockSpec((N_K, D), lambda i: (0, 0)),
        ],
        out_specs=pl.BlockSpec((BT,), lambda i: (i,)),
        out_shape=jax.ShapeDtypeStruct((n,), jnp.int32),
    )(z, emb)


def _sc_gather(emb, idx):
    n = idx.shape[0]
    idx2 = idx.reshape((1, n))

    @pl.kernel(
        out_type=jax.ShapeDtypeStruct((n, D), jnp.float32),
        mesh=plsc.VectorSubcoreMesh(
            core_axis_name="core", subcore_axis_name="subcore"),
    )
    def gather_kernel(emb_hbm, i_hbm, o_hbm):
        def body(i_vmem, o_vmem):
            pltpu.sync_copy(emb_hbm.at[i_vmem.at[0]], o_vmem)

        pltpu.emit_pipeline(
            body,
            grid=(n // GW,),
            in_specs=[pl.BlockSpec((1, GW), index_map=lambda i: (0, i))],
            out_specs=[pl.BlockSpec((GW, D), index_map=lambda i: (i, 0))],
            core_axis_name=("core", "subcore"),
            dimension_semantics=(pltpu.PARALLEL,),
        )(i_hbm, o_hbm)

    return gather_kernel(emb, idx2)


def kernel(z, emb):
    idxs, zqs = [], []
    for c in range(NCH):
        zc = jax.lax.slice_in_dim(z, c * CH, (c + 1) * CH, axis=0)
        idx_c = _tc_argmin(zc, emb)
        idxs.append(idx_c)
        zqs.append(_sc_gather(emb, idx_c))
    return (jnp.concatenate(zqs, axis=0), jnp.concatenate(idxs, axis=0))


# speedup vs baseline: 1.9201x; 1.9201x over previous
import jax
import jax.numpy as jnp
from jax.experimental import pallas as pl

N_K = 1024
D = 256
B = 16384
BT = 4096


def _vq_body(z_ref, emb_ref, zq_ref, idx_ref):
    z = z_ref[...]
    emb = emb_ref[...]
    zsq = jnp.sum(z * z, axis=-1, keepdims=True)
    esq = jnp.sum(emb * emb, axis=-1)
    zb2 = z.astype(jnp.bfloat16)
    zb2 = zb2 + zb2
    s2 = jax.lax.dot_general(
        zb2, emb.astype(jnp.bfloat16),
        (((1,), (1,)), ((), ())),
        preferred_element_type=jnp.float32)
    d = (zsq + esq[None, :]) - s2
    m = jnp.min(d, axis=-1, keepdims=True)
    iota = jax.lax.broadcasted_iota(jnp.int32, d.shape, 1)
    idx = jnp.min(jnp.where(d == m, iota, N_K), axis=-1)
    idx_ref[...] = idx
    onehot = (iota == idx[:, None]).astype(jnp.float32)
    zq_ref[...] = jax.lax.dot_general(
        onehot, emb, (((1,), (0,)), ((), ())),
        preferred_element_type=jnp.float32)


def kernel(z, emb):
    zq, idx = pl.pallas_call(
        _vq_body,
        grid=(B // BT,),
        in_specs=[
            pl.BlockSpec((BT, D), lambda i: (i, 0)),
            pl.BlockSpec((N_K, D), lambda i: (0, 0)),
        ],
        out_specs=[
            pl.BlockSpec((BT, D), lambda i: (i, 0)),
            pl.BlockSpec((BT,), lambda i: (i,)),
        ],
        out_shape=[
            jax.ShapeDtypeStruct((B, D), jnp.float32),
            jax.ShapeDtypeStruct((B,), jnp.int32),
        ],
    )(z, emb)
    return (zq, idx)


# speedup vs baseline: 2.0776x; 1.0820x over previous
import jax
import jax.numpy as jnp
from jax.experimental import pallas as pl

N_K = 1024
D = 256
B = 16384
BT = 4096


def _vq_body(z_ref, emb_ref, zq_ref, idx_ref):
    z = z_ref[...]
    emb = emb_ref[...]
    zsq = jnp.sum(z * z, axis=-1, keepdims=True)
    esq = jnp.sum(emb * emb, axis=-1)
    zb2 = z.astype(jnp.bfloat16)
    zb2 = zb2 + zb2
    s2 = jax.lax.dot_general(
        zb2, emb.astype(jnp.bfloat16),
        (((1,), (1,)), ((), ())),
        preferred_element_type=jnp.float32)
    d = (zsq + esq[None, :]) - s2
    m = jnp.min(d, axis=-1, keepdims=True)
    iota_f = jax.lax.broadcasted_iota(
        jnp.int32, d.shape, 1).astype(jnp.float32)
    idx_f = jnp.min(jnp.where(d == m, iota_f, float(N_K)), axis=-1,
                    keepdims=True)
    idx_ref[...] = idx_f[:, 0].astype(jnp.int32)
    onehot = (iota_f == idx_f).astype(jnp.float32)
    zq_ref[...] = jax.lax.dot_general(
        onehot, emb, (((1,), (0,)), ((), ())),
        preferred_element_type=jnp.float32)


def kernel(z, emb):
    zq, idx = pl.pallas_call(
        _vq_body,
        grid=(B // BT,),
        in_specs=[
            pl.BlockSpec((BT, D), lambda i: (i, 0)),
            pl.BlockSpec((N_K, D), lambda i: (0, 0)),
        ],
        out_specs=[
            pl.BlockSpec((BT, D), lambda i: (i, 0)),
            pl.BlockSpec((BT,), lambda i: (i,)),
        ],
        out_shape=[
            jax.ShapeDtypeStruct((B, D), jnp.float32),
            jax.ShapeDtypeStruct((B,), jnp.int32),
        ],
    )(z, emb)
    return (zq, idx)


# speedup vs baseline: 2.2268x; 1.0718x over previous
import jax
import jax.numpy as jnp
from jax.experimental import pallas as pl

N_K = 1024
D = 256
B = 16384
BT = 4096


def _vq_body(z_ref, emb_ref, zq_ref, idx_ref):
    z = z_ref[...]
    emb = emb_ref[...]
    zsq = jnp.sum(z * z, axis=-1, keepdims=True)
    esq = jnp.sum(emb * emb, axis=-1)
    zb2 = z.astype(jnp.bfloat16)
    zb2 = zb2 + zb2
    s2 = jax.lax.dot_general(
        zb2, emb.astype(jnp.bfloat16),
        (((1,), (1,)), ((), ())),
        preferred_element_type=jnp.float32)
    d = (zsq + esq[None, :]) - s2
    idx = jnp.argmin(d, axis=-1)
    idx_ref[...] = idx
    iota_f = jax.lax.broadcasted_iota(
        jnp.int32, d.shape, 1).astype(jnp.float32)
    idx_f = idx.astype(jnp.float32)[:, None]
    onehot = (iota_f == idx_f).astype(jnp.float32)
    zq_ref[...] = jax.lax.dot_general(
        onehot, emb, (((1,), (0,)), ((), ())),
        preferred_element_type=jnp.float32)


def kernel(z, emb):
    zq, idx = pl.pallas_call(
        _vq_body,
        grid=(B // BT,),
        in_specs=[
            pl.BlockSpec((BT, D), lambda i: (i, 0)),
            pl.BlockSpec((N_K, D), lambda i: (0, 0)),
        ],
        out_specs=[
            pl.BlockSpec((BT, D), lambda i: (i, 0)),
            pl.BlockSpec((BT,), lambda i: (i,)),
        ],
        out_shape=[
            jax.ShapeDtypeStruct((B, D), jnp.float32),
            jax.ShapeDtypeStruct((B,), jnp.int32),
        ],
    )(z, emb)
    return (zq, idx)
